# baseline (speedup 1.0000x reference)
import functools
import math

import jax
import jax.numpy as jnp
from jax import lax
from jax.experimental import pallas as pl
from jax.experimental.pallas import tpu as pltpu
from jax.experimental.pallas import tpu_sc as plsc

_NC = 2
_NS = 16
_NW = _NC * _NS
_L = 16


def _silu(x):
    return x * jax.nn.sigmoid(x)


def _matT(a, b):
    return jax.lax.dot_general(a, b, (((1,), (1,)), ((), ())),
                               preferred_element_type=jnp.float32)


def _tables_body(hs_ref, called_ref,
                 card_t_ref, seat_t_ref, role_t_ref,
                 W_trick_ref, b_trick_ref, W_simple_ref, b_simple_ref,
                 q_hand_ref, Wk_hand_ref, Wv_hand_ref,
                 q_trick_ref, Wk_trick_ref, Wv_trick_ref,
                 q_blind_ref, Wk_blind_ref, Wv_blind_ref,
                 q_bury_ref, Wk_bury_ref, Wv_bury_ref,
                 W_head_ref, b_head_ref,
                 sh_ref, sb_ref, sy_ref, stk_ref,
                 vh_ref, vt_ref, vb_ref, vy_ref, hdr_ref):
    f32 = jnp.float32
    card_table = card_t_ref[...]
    W_simple = W_simple_ref[...]
    b_simple = b_simple_ref[...]

    st = _silu(_matT(card_table, W_simple) + b_simple)
    inv_sqrt = 1.0 / math.sqrt(32.0)

    def score_v(tok, q_ref, Wk_ref, Wv_ref):
        qW = jnp.dot(q_ref[...], Wk_ref[...],
                     preferred_element_type=f32)
        score = jax.lax.dot_general(qW, tok, (((1,), (1,)), ((), ())),
                                    preferred_element_type=f32) * inv_sqrt
        v = _matT(tok, Wv_ref[...])
        return score, v

    sh, vh = score_v(st, q_hand_ref, Wk_hand_ref, Wv_hand_ref)
    sb, vb = score_v(st, q_blind_ref, Wk_blind_ref, Wv_blind_ref)
    sy, vy = score_v(st, q_bury_ref, Wk_bury_ref, Wv_bury_ref)

    ridx = jax.lax.broadcasted_iota(jnp.int32, (680, 1), 0)
    c_id = ridx % 34
    r_id = (ridx // 34) % 4
    p_id = ridx // 136
    ohc = (c_id == jax.lax.broadcasted_iota(jnp.int32, (1, 34), 1)).astype(f32)
    ohs = ((p_id + 1) == jax.lax.broadcasted_iota(jnp.int32, (1, 6), 1)).astype(f32)
    ohr = (r_id == jax.lax.broadcasted_iota(jnp.int32, (1, 4), 1)).astype(f32)
    Xc = jnp.dot(ohc, card_table, preferred_element_type=f32)
    Xs = jnp.dot(ohs, seat_t_ref[...], preferred_element_type=f32)
    Xr = jnp.dot(ohr, role_t_ref[...], preferred_element_type=f32)
    X = jnp.concatenate([Xc, Xs, Xr], axis=1)
    tt = _silu(_matT(X, W_trick_ref[...]) + b_trick_ref[...])
    stk, vt = score_v(tt, q_trick_ref, Wk_trick_ref, Wv_trick_ref)

    sh_ref[...] = sh
    sb_ref[...] = sb
    sy_ref[...] = sy
    stk_ref[...] = stk
    vh_ref[...] = vh
    vt_ref[...] = vt
    vb_ref[...] = vb
    vy_ref[...] = vy

    hcol = jax.lax.broadcasted_iota(jnp.int32, (1, 10), 1)
    inv_norm = jnp.where(hcol == 3, 1.0 / 6.0,
                         jnp.where(hcol >= 6, 0.2, 1.0)).astype(f32)
    hs = hs_ref[...] * inv_norm
    W_head = W_head_ref[...]
    ct_head = _matT(card_table, W_head[:, 10:18])
    oh_called = (called_ref[...] ==
                 jax.lax.broadcasted_iota(jnp.int32, (1, 34), 1)).astype(f32)
    hdr_ref[...] = _silu(_matT(hs, W_head[:, 0:10]) +
                         jnp.dot(oh_called, ct_head, preferred_element_type=f32) +
                         b_head_ref[...])


def _run_tables(B, header_scalar, called2, card_table, seat_table, role_table,
                W_trick, b_trick, W_simple, b_simple,
                q_hand, Wk_hand, Wv_hand, q_trick, Wk_trick, Wv_trick,
                q_blind, Wk_blind, Wv_blind, q_bury, Wk_bury, Wv_bury,
                W_head, b_head):
    BB = 4096
    f32 = jnp.float32
    grid = (B // BB,)

    def row_spec(width):
        return pl.BlockSpec((BB, width), lambda i: (i, 0))

    def full_spec(shape):
        return pl.BlockSpec(shape, lambda i: tuple(0 for _ in shape))

    args = (header_scalar, called2,
            card_table, seat_table, role_table,
            W_trick, b_trick.reshape(1, -1), W_simple, b_simple.reshape(1, -1),
            q_hand.reshape(1, -1), Wk_hand, Wv_hand,
            q_trick.reshape(1, -1), Wk_trick, Wv_trick,
            q_blind.reshape(1, -1), Wk_blind, Wv_blind,
            q_bury.reshape(1, -1), Wk_bury, Wv_bury,
            W_head, b_head.reshape(1, -1))
    in_specs = [row_spec(10), row_spec(1)]
    in_specs += [full_spec(a.shape) for a in args[2:]]

    out_shape = (jax.ShapeDtypeStruct((1, 34), f32),
                 jax.ShapeDtypeStruct((1, 34), f32),
                 jax.ShapeDtypeStruct((1, 34), f32),
                 jax.ShapeDtypeStruct((1, 680), f32),
                 jax.ShapeDtypeStruct((34, 64), f32),
                 jax.ShapeDtypeStruct((680, 64), f32),
                 jax.ShapeDtypeStruct((34, 32), f32),
                 jax.ShapeDtypeStruct((34, 32), f32),
                 jax.ShapeDtypeStruct((B, 64), f32))
    out_specs = [full_spec((1, 34)), full_spec((1, 34)), full_spec((1, 34)),
                 full_spec((1, 680)), full_spec((34, 64)), full_spec((680, 64)),
                 full_spec((34, 32)), full_spec((34, 32)),
                 pl.BlockSpec((BB, 64), lambda i: (i, 0))]

    return pl.pallas_call(
        _tables_body, grid=grid, in_specs=in_specs,
        out_specs=out_specs, out_shape=out_shape)(*args)


def _make_sc(B):
    RPT = B // _NW
    CH = 64
    NCH = RPT // CH
    GPC = CH // _L
    i32, f32 = jnp.int32, jnp.float32
    mesh = plsc.VectorSubcoreMesh(core_axis_name="c", subcore_axis_name="s")

    O_TC, O_TP, O_TPT = B * 8, B * 13, B * 18
    O_BL, O_BY, O_HDR = B * 23, B * 25, B * 27
    T_SB, T_SY, T_STK = 40, 80, 120
    T_VH, T_VT = 800, 1928
    T_VB, T_VY = 24368, 24952

    @functools.partial(
        pl.kernel,
        out_type=jax.ShapeDtypeStruct((B * 256,), f32),
        mesh=mesh,
        compiler_params=pltpu.CompilerParams(needs_layout_passes=False),
        scratch_types=[
            pltpu.VMEM((RPT * 8,), i32),
            pltpu.VMEM((RPT * 5,), i32),
            pltpu.VMEM((RPT * 5,), i32),
            pltpu.VMEM((RPT * 5,), i32),
            pltpu.VMEM((RPT * 2,), i32),
            pltpu.VMEM((RPT * 2,), i32),
            pltpu.VMEM((34,), i32),
            pltpu.VMEM((34,), i32),
            pltpu.VMEM((34,), i32),
            pltpu.VMEM((680,), i32),
            pltpu.VMEM((34 * 33,), i32),
            pltpu.VMEM((680 * 33,), i32),
            pltpu.VMEM((34 * 17,), i32),
            pltpu.VMEM((34 * 17,), i32),
            pltpu.VMEM((CH * 64,), i32),
            pltpu.VMEM((12 * 272,), f32),
            pltpu.VMEM((CH * 256,), f32),
            pltpu.SemaphoreType.DMA,
            pltpu.SemaphoreType.DMA,
        ],
    )
    def sc_fn(ids_h, tab_h, out_h,
              hand_v, tc_v, tp_v, tpt_v, bl_v, by_v,
              sh_v, sb_v, sy_v, stk_v, vh_v, vt_v, vb_v, vy_v,
              hdr_v, tr_v, out_v, sem, sem_h):
        wid = lax.axis_index("s") * _NC + lax.axis_index("c")
        base = wid * RPT
        copies = [
            pltpu.async_copy(ids_h.at[pl.ds(base * 8, RPT * 8)], hand_v, sem),
            pltpu.async_copy(ids_h.at[pl.ds(O_TC + base * 5, RPT * 5)], tc_v, sem),
            pltpu.async_copy(ids_h.at[pl.ds(O_TP + base * 5, RPT * 5)], tp_v, sem),
            pltpu.async_copy(ids_h.at[pl.ds(O_TPT + base * 5, RPT * 5)], tpt_v, sem),
            pltpu.async_copy(ids_h.at[pl.ds(O_BL + base * 2, RPT * 2)], bl_v, sem),
            pltpu.async_copy(ids_h.at[pl.ds(O_BY + base * 2, RPT * 2)], by_v, sem),
            pltpu.async_copy(tab_h.at[pl.ds(0, 34)], sh_v, sem),
            pltpu.async_copy(tab_h.at[pl.ds(T_SB, 34)], sb_v, sem),
            pltpu.async_copy(tab_h.at[pl.ds(T_SY, 34)], sy_v, sem),
            pltpu.async_copy(tab_h.at[pl.ds(T_STK, 680)], stk_v, sem),
            pltpu.async_copy(tab_h.at[pl.ds(T_VH, 34 * 33)], vh_v, sem),
            pltpu.async_copy(tab_h.at[pl.ds(T_VT, 680 * 33)], vt_v, sem),
            pltpu.async_copy(tab_h.at[pl.ds(T_VB, 34 * 17)], vb_v, sem),
            pltpu.async_copy(tab_h.at[pl.ds(T_VY, 34 * 17)], vy_v, sem),
        ]
        for c in copies:
            c.wait()

        iota = lax.iota(i32, _L)
        NEG = -1000000000.0

        def pool_write(tix, masks, stab, vtab, stride, Dv, col0, lbase,
                       rbase0):
            P = len(tix)
            s = [plsc.bitcast(plsc.load_gather(stab, [tix[j]]), f32)
                 for j in range(P)]
            att = [jnp.where(masks[j], s[j], NEG) for j in range(P)]
            m = att[0]
            for j in range(1, P):
                m = jnp.maximum(m, att[j])
            e = [jnp.exp(att[j] - m) for j in range(P)]
            z = e[0]
            for j in range(1, P):
                z = z + e[j]
            anyv = masks[0]
            for j in range(1, P):
                anyv = anyv | masks[j]
            inv = jnp.where(anyv, 1.0 / z, 0.0)
            w = [e[j] * inv for j in range(P)]
            vbase = [tix[j] * stride for j in range(P)]
            HI = jnp.int32(-65536)

            @plsc.parallel_loop(0, Dv // 16, 1, unroll=2)
            def _dchunk(it):
                wc = it * 8
                rbase = (rbase0 + it) * 272
                accs = [None] * 16
                for j in range(P):
                    for k in range(8):
                        word = plsc.load_gather(vtab, [vbase[j] + (wc + k)])
                        lo = plsc.bitcast(word << 16, f32)
                        hi = plsc.bitcast(word & HI, f32)
                        if accs[2 * k] is None:
                            accs[2 * k] = w[0] * lo
                            accs[2 * k + 1] = w[0] * hi
                        else:
                            accs[2 * k] = accs[2 * k] + w[j] * lo
                            accs[2 * k + 1] = accs[2 * k + 1] + w[j] * hi
                for d in range(16):
                    plsc.store_scatter(tr_v, [rbase + iota * 17 + d], accs[d])
                for k in range(16):
                    out_v[pl.ds((lbase + k) * 256 + col0 + it * 16, 16)] = (
                        tr_v[pl.ds(rbase + k * 17, 16)])

        def do_group(gi, crow):
            rows = crow + gi * _L + iota
            lbase = gi * _L

            rb = rows * 8
            ids = [plsc.load_gather(hand_v, [rb + j]) for j in range(8)]
            masks = [idj != 0 for idj in ids]
            pool_write(ids, masks, sh_v, vh_v, 33, 64, 0, lbase, 0)

            rb = rows * 5
            cards = [plsc.load_gather(tc_v, [rb + j]) for j in range(5)]
            picks = [plsc.load_gather(tp_v, [rb + j]) for j in range(5)]
            parts = [plsc.load_gather(tpt_v, [rb + j]) for j in range(5)]
            tix = [cards[j] + 34 * (picks[j] + 2 * parts[j]) + 136 * j
                   for j in range(5)]
            masks = [cj != 0 for cj in cards]
            pool_write(tix, masks, stk_v, vt_v, 33, 64, 64, lbase, 4)

            rb = rows * 2
            ids = [plsc.load_gather(bl_v, [rb + j]) for j in range(2)]
            masks = [idj != 0 for idj in ids]
            pool_write(ids, masks, sb_v, vb_v, 17, 32, 128, lbase, 8)
            ids = [plsc.load_gather(by_v, [rb + j]) for j in range(2)]
            masks = [idj != 0 for idj in ids]
            pool_write(ids, masks, sy_v, vy_v, 17, 32, 160, lbase, 10)

            @plsc.parallel_loop(0, 16, 1, unroll=4)
            def _hrow(k):
                for dc in range(0, 64, 16):
                    out_v[pl.ds((lbase + k) * 256 + 192 + dc, 16)] = (
                        plsc.bitcast(hdr_v[pl.ds((lbase + k) * 64 + dc, 16)],
                                     f32))
            return crow

        def do_chunk(ci, carry):
            crow = ci * CH
            hcopy = pltpu.async_copy(
                ids_h.at[pl.ds(O_HDR + (base + crow) * 64, CH * 64)],
                hdr_v, sem_h)
            hcopy.wait()
            lax.fori_loop(0, GPC, do_group, crow)
            pltpu.sync_copy(out_v,
                            out_h.at[pl.ds((base + crow) * 256, CH * 256)])
            return carry

        lax.fori_loop(0, NCH, do_chunk, 0)

    return sc_fn


def kernel(header_scalar, called_ids, hand_ids, blind_ids, bury_ids,
           trick_card_ids, trick_is_picker, trick_is_partner_known,
           card_table, seat_table, role_table,
           W_trick, b_trick, W_simple, b_simple,
           q_hand, Wk_hand, Wv_hand, q_trick, Wk_trick, Wv_trick,
           q_blind, Wk_blind, Wv_blind, q_bury, Wk_bury, Wv_bury,
           W_head, b_head):
    B = header_scalar.shape[0]
    i32 = jnp.int32

    called2 = called_ids.reshape(B, 1).astype(i32)
    sh, sb, sy, stk, vh, vt, vb, vy, hdr = _run_tables(
        B, header_scalar, called2, card_table, seat_table, role_table,
        W_trick, b_trick, W_simple, b_simple,
        q_hand, Wk_hand, Wv_hand, q_trick, Wk_trick, Wv_trick,
        q_blind, Wk_blind, Wv_blind, q_bury, Wk_bury, Wv_bury,
        W_head, b_head)

    def bc(x):
        return jax.lax.bitcast_convert_type(x, i32)

    def pack_v(v):
        T, Dv = v.shape
        w = jax.lax.bitcast_convert_type(
            v.astype(jnp.bfloat16).reshape(T, Dv // 2, 2), i32)
        return jnp.pad(w, ((0, 0), (0, 1))).reshape(-1)

    def sec(x, L):
        return jnp.pad(x, (0, L - x.shape[0]))

    tab_blob = jnp.concatenate([
        sec(bc(sh.reshape(-1)), 40),
        sec(bc(sb.reshape(-1)), 40),
        sec(bc(sy.reshape(-1)), 40),
        bc(stk.reshape(-1)),
        sec(pack_v(vh), 1128),
        pack_v(vt),
        sec(pack_v(vb), 584),
        sec(pack_v(vy), 584),
    ])
    ids_blob = jnp.concatenate([
        hand_ids.astype(i32).reshape(-1),
        trick_card_ids.astype(i32).reshape(-1),
        trick_is_picker.astype(i32).reshape(-1),
        trick_is_partner_known.astype(i32).reshape(-1),
        blind_ids.astype(i32).reshape(-1),
        bury_ids.astype(i32).reshape(-1),
        bc(hdr.reshape(-1)),
    ])
    out_flat = _make_sc(B)(ids_blob, tab_blob)
    return out_flat.reshape(B, 256)


# speedup vs baseline: 1.3157x; 1.3157x over previous
import functools
import math

import jax
import jax.numpy as jnp
from jax import lax
from jax.experimental import pallas as pl
from jax.experimental.pallas import tpu as pltpu
from jax.experimental.pallas import tpu_sc as plsc

_NC = 2
_NS = 16
_NW = _NC * _NS
_L = 16


def _silu(x):
    return x * jax.nn.sigmoid(x)


def _matT(a, b):
    return jax.lax.dot_general(a, b, (((1,), (1,)), ((), ())),
                               preferred_element_type=jnp.float32)


def _tables_body(hs_ref, called_ref,
                 card_t_ref, seat_t_ref, role_t_ref,
                 W_trick_ref, b_trick_ref, W_simple_ref, b_simple_ref,
                 q_hand_ref, Wk_hand_ref, Wv_hand_ref,
                 q_trick_ref, Wk_trick_ref, Wv_trick_ref,
                 q_blind_ref, Wk_blind_ref, Wv_blind_ref,
                 q_bury_ref, Wk_bury_ref, Wv_bury_ref,
                 W_head_ref, b_head_ref,
                 sh_ref, sb_ref, sy_ref, stk_ref,
                 vh_ref, vt_ref, vb_ref, vy_ref, hdr_ref):
    f32 = jnp.float32
    card_table = card_t_ref[...]
    W_simple = W_simple_ref[...]
    b_simple = b_simple_ref[...]

    st = _silu(_matT(card_table, W_simple) + b_simple)
    inv_sqrt = 1.0 / math.sqrt(32.0)

    def score_v(tok, q_ref, Wk_ref, Wv_ref):
        qW = jnp.dot(q_ref[...], Wk_ref[...],
                     preferred_element_type=f32)
        score = jax.lax.dot_general(qW, tok, (((1,), (1,)), ((), ())),
                                    preferred_element_type=f32) * inv_sqrt
        v = _matT(tok, Wv_ref[...])
        return score, v

    sh, vh = score_v(st, q_hand_ref, Wk_hand_ref, Wv_hand_ref)
    sb, vb = score_v(st, q_blind_ref, Wk_blind_ref, Wv_blind_ref)
    sy, vy = score_v(st, q_bury_ref, Wk_bury_ref, Wv_bury_ref)

    ridx = jax.lax.broadcasted_iota(jnp.int32, (680, 1), 0)
    c_id = ridx % 34
    r_id = (ridx // 34) % 4
    p_id = ridx // 136
    ohc = (c_id == jax.lax.broadcasted_iota(jnp.int32, (1, 34), 1)).astype(f32)
    ohs = ((p_id + 1) == jax.lax.broadcasted_iota(jnp.int32, (1, 6), 1)).astype(f32)
    ohr = (r_id == jax.lax.broadcasted_iota(jnp.int32, (1, 4), 1)).astype(f32)
    Xc = jnp.dot(ohc, card_table, preferred_element_type=f32)
    Xs = jnp.dot(ohs, seat_t_ref[...], preferred_element_type=f32)
    Xr = jnp.dot(ohr, role_t_ref[...], preferred_element_type=f32)
    X = jnp.concatenate([Xc, Xs, Xr], axis=1)
    tt = _silu(_matT(X, W_trick_ref[...]) + b_trick_ref[...])
    stk, vt = score_v(tt, q_trick_ref, Wk_trick_ref, Wv_trick_ref)

    sh_ref[...] = sh
    sb_ref[...] = sb
    sy_ref[...] = sy
    stk_ref[...] = stk
    vh_ref[...] = vh
    vt_ref[...] = vt
    vb_ref[...] = vb
    vy_ref[...] = vy

    hcol = jax.lax.broadcasted_iota(jnp.int32, (1, 10), 1)
    inv_norm = jnp.where(hcol == 3, 1.0 / 6.0,
                         jnp.where(hcol >= 6, 0.2, 1.0)).astype(f32)
    hs = hs_ref[...] * inv_norm
    W_head = W_head_ref[...]
    ct_head = _matT(card_table, W_head[:, 10:18])
    oh_called = (called_ref[...] ==
                 jax.lax.broadcasted_iota(jnp.int32, (1, 34), 1)).astype(f32)
    hdr_ref[...] = _silu(_matT(hs, W_head[:, 0:10]) +
                         jnp.dot(oh_called, ct_head, preferred_element_type=f32) +
                         b_head_ref[...])


def _run_tables(B, header_scalar, called2, card_table, seat_table, role_table,
                W_trick, b_trick, W_simple, b_simple,
                q_hand, Wk_hand, Wv_hand, q_trick, Wk_trick, Wv_trick,
                q_blind, Wk_blind, Wv_blind, q_bury, Wk_bury, Wv_bury,
                W_head, b_head):
    BB = 4096
    f32 = jnp.float32
    grid = (B // BB,)

    def row_spec(width):
        return pl.BlockSpec((BB, width), lambda i: (i, 0))

    def full_spec(shape):
        return pl.BlockSpec(shape, lambda i: tuple(0 for _ in shape))

    args = (header_scalar, called2,
            card_table, seat_table, role_table,
            W_trick, b_trick.reshape(1, -1), W_simple, b_simple.reshape(1, -1),
            q_hand.reshape(1, -1), Wk_hand, Wv_hand,
            q_trick.reshape(1, -1), Wk_trick, Wv_trick,
            q_blind.reshape(1, -1), Wk_blind, Wv_blind,
            q_bury.reshape(1, -1), Wk_bury, Wv_bury,
            W_head, b_head.reshape(1, -1))
    in_specs = [row_spec(10), row_spec(1)]
    in_specs += [full_spec(a.shape) for a in args[2:]]

    out_shape = (jax.ShapeDtypeStruct((1, 34), f32),
                 jax.ShapeDtypeStruct((1, 34), f32),
                 jax.ShapeDtypeStruct((1, 34), f32),
                 jax.ShapeDtypeStruct((1, 680), f32),
                 jax.ShapeDtypeStruct((34, 64), f32),
                 jax.ShapeDtypeStruct((680, 64), f32),
                 jax.ShapeDtypeStruct((34, 32), f32),
                 jax.ShapeDtypeStruct((34, 32), f32),
                 jax.ShapeDtypeStruct((B, 64), f32))
    out_specs = [full_spec((1, 34)), full_spec((1, 34)), full_spec((1, 34)),
                 full_spec((1, 680)), full_spec((34, 64)), full_spec((680, 64)),
                 full_spec((34, 32)), full_spec((34, 32)),
                 pl.BlockSpec((BB, 64), lambda i: (i, 0))]

    return pl.pallas_call(
        _tables_body, grid=grid, in_specs=in_specs,
        out_specs=out_specs, out_shape=out_shape)(*args)


def _make_sc(B):
    RPT = B // _NW
    CH = 64
    NCH = RPT // CH
    GPC = CH // _L
    i32, f32 = jnp.int32, jnp.float32
    mesh = plsc.VectorSubcoreMesh(core_axis_name="c", subcore_axis_name="s")

    T_SB, T_SY, T_STK = 40, 80, 120
    T_VH, T_VT = 800, 1928
    T_VB, T_VY = 24368, 24952

    @functools.partial(
        pl.kernel,
        out_type=jax.ShapeDtypeStruct((B * 256,), f32),
        mesh=mesh,
        compiler_params=pltpu.CompilerParams(needs_layout_passes=False),
        scratch_types=[
            pltpu.VMEM((RPT * 27,), i32),
            pltpu.VMEM((34,), i32),
            pltpu.VMEM((34,), i32),
            pltpu.VMEM((34,), i32),
            pltpu.VMEM((680,), i32),
            pltpu.VMEM((34 * 33,), i32),
            pltpu.VMEM((680 * 33,), i32),
            pltpu.VMEM((34 * 17,), i32),
            pltpu.VMEM((34 * 17,), i32),
            pltpu.VMEM((CH * 64,), i32),
            pltpu.VMEM((12 * 272,), f32),
            pltpu.VMEM((CH * 256,), f32),
            pltpu.SemaphoreType.DMA,
            pltpu.SemaphoreType.DMA,
        ],
    )
    def sc_fn(ids_h, hdr_h, tab_h, out_h,
              ids_v,
              sh_v, sb_v, sy_v, stk_v, vh_v, vt_v, vb_v, vy_v,
              hdr_v, tr_v, out_v, sem, sem_h):
        wid = lax.axis_index("s") * _NC + lax.axis_index("c")
        base = wid * RPT
        copies = [
            pltpu.async_copy(ids_h.at[pl.ds(base * 27, RPT * 27)], ids_v, sem),
            pltpu.async_copy(tab_h.at[pl.ds(0, 34)], sh_v, sem),
            pltpu.async_copy(tab_h.at[pl.ds(T_SB, 34)], sb_v, sem),
            pltpu.async_copy(tab_h.at[pl.ds(T_SY, 34)], sy_v, sem),
            pltpu.async_copy(tab_h.at[pl.ds(T_STK, 680)], stk_v, sem),
            pltpu.async_copy(tab_h.at[pl.ds(T_VH, 34 * 33)], vh_v, sem),
            pltpu.async_copy(tab_h.at[pl.ds(T_VT, 680 * 33)], vt_v, sem),
            pltpu.async_copy(tab_h.at[pl.ds(T_VB, 34 * 17)], vb_v, sem),
            pltpu.async_copy(tab_h.at[pl.ds(T_VY, 34 * 17)], vy_v, sem),
        ]
        for c in copies:
            c.wait()

        iota = lax.iota(i32, _L)
        NEG = -1000000000.0

        def pool_write(tix, masks, stab, vtab, stride, Dv, col0, lbase,
                       rbase0):
            P = len(tix)
            s = [plsc.bitcast(plsc.load_gather(stab, [tix[j]]), f32)
                 for j in range(P)]
            att = [jnp.where(masks[j], s[j], NEG) for j in range(P)]
            m = att[0]
            for j in range(1, P):
                m = jnp.maximum(m, att[j])
            e = [jnp.exp(att[j] - m) for j in range(P)]
            z = e[0]
            for j in range(1, P):
                z = z + e[j]
            anyv = masks[0]
            for j in range(1, P):
                anyv = anyv | masks[j]
            inv = jnp.where(anyv, 1.0 / z, 0.0)
            w = [e[j] * inv for j in range(P)]
            vbase = [tix[j] * stride for j in range(P)]
            HI = jnp.int32(-65536)

            @plsc.parallel_loop(0, Dv // 16, 1, unroll=2)
            def _dchunk(it):
                wc = it * 8
                rbase = (rbase0 + it) * 272
                accs = [None] * 16
                for j in range(P):
                    for k in range(8):
                        word = plsc.load_gather(vtab, [vbase[j] + (wc + k)])
                        lo = plsc.bitcast(word << 16, f32)
                        hi = plsc.bitcast(word & HI, f32)
                        if accs[2 * k] is None:
                            accs[2 * k] = w[0] * lo
                            accs[2 * k + 1] = w[0] * hi
                        else:
                            accs[2 * k] = accs[2 * k] + w[j] * lo
                            accs[2 * k + 1] = accs[2 * k + 1] + w[j] * hi
                for d in range(16):
                    plsc.store_scatter(tr_v, [rbase + iota * 17 + d], accs[d])
                for k in range(16):
                    out_v[pl.ds((lbase + k) * 256 + col0 + it * 16, 16)] = (
                        tr_v[pl.ds(rbase + k * 17, 16)])

        def do_group(gi, crow):
            rows = crow + gi * _L + iota
            lbase = gi * _L

            rb = rows * 27
            ids = [plsc.load_gather(ids_v, [rb + j]) for j in range(8)]
            masks = [idj != 0 for idj in ids]
            pool_write(ids, masks, sh_v, vh_v, 33, 64, 0, lbase, 0)

            cards = [plsc.load_gather(ids_v, [rb + 8 + j]) for j in range(5)]
            picks = [plsc.load_gather(ids_v, [rb + 13 + j]) for j in range(5)]
            parts = [plsc.load_gather(ids_v, [rb + 18 + j]) for j in range(5)]
            tix = [cards[j] + 34 * (picks[j] + 2 * parts[j]) + 136 * j
                   for j in range(5)]
            masks = [cj != 0 for cj in cards]
            pool_write(tix, masks, stk_v, vt_v, 33, 64, 64, lbase, 4)

            ids = [plsc.load_gather(ids_v, [rb + 23 + j]) for j in range(2)]
            masks = [idj != 0 for idj in ids]
            pool_write(ids, masks, sb_v, vb_v, 17, 32, 128, lbase, 8)
            ids = [plsc.load_gather(ids_v, [rb + 25 + j]) for j in range(2)]
            masks = [idj != 0 for idj in ids]
            pool_write(ids, masks, sy_v, vy_v, 17, 32, 160, lbase, 10)

            @plsc.parallel_loop(0, 16, 1, unroll=4)
            def _hrow(k):
                for dc in range(0, 64, 16):
                    out_v[pl.ds((lbase + k) * 256 + 192 + dc, 16)] = (
                        plsc.bitcast(hdr_v[pl.ds((lbase + k) * 64 + dc, 16)],
                                     f32))
            return crow

        def do_chunk(ci, carry):
            crow = ci * CH
            hcopy = pltpu.async_copy(
                hdr_h.at[pl.ds((base + crow) * 64, CH * 64)], hdr_v, sem_h)
            hcopy.wait()
            lax.fori_loop(0, GPC, do_group, crow)
            pltpu.sync_copy(out_v,
                            out_h.at[pl.ds((base + crow) * 256, CH * 256)])
            return carry

        lax.fori_loop(0, NCH, do_chunk, 0)

    return sc_fn


def kernel(header_scalar, called_ids, hand_ids, blind_ids, bury_ids,
           trick_card_ids, trick_is_picker, trick_is_partner_known,
           card_table, seat_table, role_table,
           W_trick, b_trick, W_simple, b_simple,
           q_hand, Wk_hand, Wv_hand, q_trick, Wk_trick, Wv_trick,
           q_blind, Wk_blind, Wv_blind, q_bury, Wk_bury, Wv_bury,
           W_head, b_head):
    B = header_scalar.shape[0]
    i32 = jnp.int32

    called2 = called_ids.reshape(B, 1).astype(i32)
    sh, sb, sy, stk, vh, vt, vb, vy, hdr = _run_tables(
        B, header_scalar, called2, card_table, seat_table, role_table,
        W_trick, b_trick, W_simple, b_simple,
        q_hand, Wk_hand, Wv_hand, q_trick, Wk_trick, Wv_trick,
        q_blind, Wk_blind, Wv_blind, q_bury, Wk_bury, Wv_bury,
        W_head, b_head)

    def bc(x):
        return jax.lax.bitcast_convert_type(x, i32)

    def pack_v(v):
        T, Dv = v.shape
        w = jax.lax.bitcast_convert_type(
            v.astype(jnp.bfloat16).reshape(T, Dv // 2, 2), i32)
        return jnp.pad(w, ((0, 0), (0, 1))).reshape(-1)

    def sec(x, L):
        return jnp.pad(x, (0, L - x.shape[0]))

    tab_blob = jnp.concatenate([
        sec(bc(sh.reshape(-1)), 40),
        sec(bc(sb.reshape(-1)), 40),
        sec(bc(sy.reshape(-1)), 40),
        bc(stk.reshape(-1)),
        sec(pack_v(vh), 1128),
        pack_v(vt),
        sec(pack_v(vb), 584),
        sec(pack_v(vy), 584),
    ])
    ids27 = jnp.concatenate(
        [hand_ids, trick_card_ids, trick_is_picker, trick_is_partner_known,
         blind_ids, bury_ids], axis=1).astype(i32)
    out_flat = _make_sc(B)(ids27.reshape(-1), bc(hdr.reshape(-1)), tab_blob)
    return out_flat.reshape(B, 256)


# speedup vs baseline: 1.3544x; 1.0294x over previous
import functools
import math

import jax
import jax.numpy as jnp
from jax import lax
from jax.experimental import pallas as pl
from jax.experimental.pallas import tpu as pltpu
from jax.experimental.pallas import tpu_sc as plsc

_NC = 2
_NS = 16
_NW = _NC * _NS
_L = 16


def _silu(x):
    return x * jax.nn.sigmoid(x)


def _matT(a, b):
    return jax.lax.dot_general(a, b, (((1,), (1,)), ((), ())),
                               preferred_element_type=jnp.float32)


def _tables_body(hs_ref, called_ref,
                 card_t_ref, seat_t_ref, role_t_ref,
                 W_trick_ref, b_trick_ref, W_simple_ref, b_simple_ref,
                 q_hand_ref, Wk_hand_ref, Wv_hand_ref,
                 q_trick_ref, Wk_trick_ref, Wv_trick_ref,
                 q_blind_ref, Wk_blind_ref, Wv_blind_ref,
                 q_bury_ref, Wk_bury_ref, Wv_bury_ref,
                 W_head_ref, b_head_ref,
                 sh_ref, sb_ref, sy_ref, stk_ref,
                 vh_ref, vt_ref, vb_ref, vy_ref, hdr_ref):
    f32 = jnp.float32
    card_table = card_t_ref[...]
    W_simple = W_simple_ref[...]
    b_simple = b_simple_ref[...]

    st = _silu(_matT(card_table, W_simple) + b_simple)
    inv_sqrt = 1.0 / math.sqrt(32.0)

    def score_v(tok, q_ref, Wk_ref, Wv_ref):
        qW = jnp.dot(q_ref[...], Wk_ref[...],
                     preferred_element_type=f32)
        score = jax.lax.dot_general(qW, tok, (((1,), (1,)), ((), ())),
                                    preferred_element_type=f32) * inv_sqrt
        v = _matT(tok, Wv_ref[...])
        return score, v

    sh, vh = score_v(st, q_hand_ref, Wk_hand_ref, Wv_hand_ref)
    sb, vb = score_v(st, q_blind_ref, Wk_blind_ref, Wv_blind_ref)
    sy, vy = score_v(st, q_bury_ref, Wk_bury_ref, Wv_bury_ref)

    ridx = jax.lax.broadcasted_iota(jnp.int32, (680, 1), 0)
    c_id = ridx % 34
    r_id = (ridx // 34) % 4
    p_id = ridx // 136
    ohc = (c_id == jax.lax.broadcasted_iota(jnp.int32, (1, 34), 1)).astype(f32)
    ohs = ((p_id + 1) == jax.lax.broadcasted_iota(jnp.int32, (1, 6), 1)).astype(f32)
    ohr = (r_id == jax.lax.broadcasted_iota(jnp.int32, (1, 4), 1)).astype(f32)
    Xc = jnp.dot(ohc, card_table, preferred_element_type=f32)
    Xs = jnp.dot(ohs, seat_t_ref[...], preferred_element_type=f32)
    Xr = jnp.dot(ohr, role_t_ref[...], preferred_element_type=f32)
    X = jnp.concatenate([Xc, Xs, Xr], axis=1)
    tt = _silu(_matT(X, W_trick_ref[...]) + b_trick_ref[...])
    stk, vt = score_v(tt, q_trick_ref, Wk_trick_ref, Wv_trick_ref)

    sh_ref[...] = sh
    sb_ref[...] = sb
    sy_ref[...] = sy
    stk_ref[...] = stk
    vh_ref[...] = vh
    vt_ref[...] = vt
    vb_ref[...] = vb
    vy_ref[...] = vy

    hcol = jax.lax.broadcasted_iota(jnp.int32, (1, 10), 1)
    inv_norm = jnp.where(hcol == 3, 1.0 / 6.0,
                         jnp.where(hcol >= 6, 0.2, 1.0)).astype(f32)
    hs = hs_ref[...] * inv_norm
    W_head = W_head_ref[...]
    ct_head = _matT(card_table, W_head[:, 10:18])
    oh_called = (called_ref[...] ==
                 jax.lax.broadcasted_iota(jnp.int32, (1, 34), 1)).astype(f32)
    hdr_ref[...] = _silu(_matT(hs, W_head[:, 0:10]) +
                         jnp.dot(oh_called, ct_head, preferred_element_type=f32) +
                         b_head_ref[...])


def _run_tables(B, header_scalar, called2, card_table, seat_table, role_table,
                W_trick, b_trick, W_simple, b_simple,
                q_hand, Wk_hand, Wv_hand, q_trick, Wk_trick, Wv_trick,
                q_blind, Wk_blind, Wv_blind, q_bury, Wk_bury, Wv_bury,
                W_head, b_head):
    BB = 4096
    f32 = jnp.float32
    grid = (B // BB,)

    def row_spec(width):
        return pl.BlockSpec((BB, width), lambda i: (i, 0))

    def full_spec(shape):
        return pl.BlockSpec(shape, lambda i: tuple(0 for _ in shape))

    args = (header_scalar, called2,
            card_table, seat_table, role_table,
            W_trick, b_trick.reshape(1, -1), W_simple, b_simple.reshape(1, -1),
            q_hand.reshape(1, -1), Wk_hand, Wv_hand,
            q_trick.reshape(1, -1), Wk_trick, Wv_trick,
            q_blind.reshape(1, -1), Wk_blind, Wv_blind,
            q_bury.reshape(1, -1), Wk_bury, Wv_bury,
            W_head, b_head.reshape(1, -1))
    in_specs = [row_spec(10), row_spec(1)]
    in_specs += [full_spec(a.shape) for a in args[2:]]

    out_shape = (jax.ShapeDtypeStruct((1, 34), f32),
                 jax.ShapeDtypeStruct((1, 34), f32),
                 jax.ShapeDtypeStruct((1, 34), f32),
                 jax.ShapeDtypeStruct((1, 680), f32),
                 jax.ShapeDtypeStruct((34, 64), f32),
                 jax.ShapeDtypeStruct((680, 64), f32),
                 jax.ShapeDtypeStruct((34, 32), f32),
                 jax.ShapeDtypeStruct((34, 32), f32),
                 jax.ShapeDtypeStruct((B, 64), f32))
    out_specs = [full_spec((1, 34)), full_spec((1, 34)), full_spec((1, 34)),
                 full_spec((1, 680)), full_spec((34, 64)), full_spec((680, 64)),
                 full_spec((34, 32)), full_spec((34, 32)),
                 pl.BlockSpec((BB, 64), lambda i: (i, 0))]

    return pl.pallas_call(
        _tables_body, grid=grid, in_specs=in_specs,
        out_specs=out_specs, out_shape=out_shape)(*args)


def _make_sc(B):
    RPT = B // _NW
    CH = 128
    NCH = RPT // CH
    GPC = CH // _L
    i32, f32 = jnp.int32, jnp.float32
    mesh = plsc.VectorSubcoreMesh(core_axis_name="c", subcore_axis_name="s")

    T_SB, T_SY, T_STK = 40, 80, 120
    T_VH, T_VT = 800, 1928
    T_VB, T_VY = 24368, 24952

    @functools.partial(
        pl.kernel,
        out_type=jax.ShapeDtypeStruct((B * 256,), f32),
        mesh=mesh,
        compiler_params=pltpu.CompilerParams(needs_layout_passes=False),
        scratch_types=[
            pltpu.VMEM((RPT * 27,), i32),
            pltpu.VMEM((34,), i32),
            pltpu.VMEM((34,), i32),
            pltpu.VMEM((34,), i32),
            pltpu.VMEM((680,), i32),
            pltpu.VMEM((34 * 33,), i32),
            pltpu.VMEM((680 * 33,), i32),
            pltpu.VMEM((34 * 17,), i32),
            pltpu.VMEM((34 * 17,), i32),
            pltpu.VMEM((CH * 64,), i32),
            pltpu.VMEM((12 * 272,), f32),
            pltpu.VMEM((CH * 256,), f32),
            pltpu.SemaphoreType.DMA,
            pltpu.SemaphoreType.DMA,
        ],
    )
    def sc_fn(ids_h, hdr_h, tab_h, out_h,
              ids_v,
              sh_v, sb_v, sy_v, stk_v, vh_v, vt_v, vb_v, vy_v,
              hdr_v, tr_v, out_v, sem, sem_h):
        wid = lax.axis_index("s") * _NC + lax.axis_index("c")
        base = wid * RPT
        copies = [
            pltpu.async_copy(ids_h.at[pl.ds(base * 27, RPT * 27)], ids_v, sem),
            pltpu.async_copy(tab_h.at[pl.ds(0, 34)], sh_v, sem),
            pltpu.async_copy(tab_h.at[pl.ds(T_SB, 34)], sb_v, sem),
            pltpu.async_copy(tab_h.at[pl.ds(T_SY, 34)], sy_v, sem),
            pltpu.async_copy(tab_h.at[pl.ds(T_STK, 680)], stk_v, sem),
            pltpu.async_copy(tab_h.at[pl.ds(T_VH, 34 * 33)], vh_v, sem),
            pltpu.async_copy(tab_h.at[pl.ds(T_VT, 680 * 33)], vt_v, sem),
            pltpu.async_copy(tab_h.at[pl.ds(T_VB, 34 * 17)], vb_v, sem),
            pltpu.async_copy(tab_h.at[pl.ds(T_VY, 34 * 17)], vy_v, sem),
        ]
        for c in copies:
            c.wait()

        iota = lax.iota(i32, _L)
        NEG = -1000000000.0

        def pool_write(tix, masks, stab, vtab, stride, Dv, col0, lbase,
                       rbase0):
            P = len(tix)
            s = [plsc.bitcast(plsc.load_gather(stab, [tix[j]]), f32)
                 for j in range(P)]
            att = [jnp.where(masks[j], s[j], NEG) for j in range(P)]
            m = att[0]
            for j in range(1, P):
                m = jnp.maximum(m, att[j])
            e = [jnp.exp(att[j] - m) for j in range(P)]
            z = e[0]
            for j in range(1, P):
                z = z + e[j]
            anyv = masks[0]
            for j in range(1, P):
                anyv = anyv | masks[j]
            inv = jnp.where(anyv, 1.0 / z, 0.0)
            w = [e[j] * inv for j in range(P)]
            vbase = [tix[j] * stride for j in range(P)]
            HI = jnp.int32(-65536)

            @plsc.parallel_loop(0, Dv // 16, 1, unroll=2)
            def _dchunk(it):
                wc = it * 8
                rbase = (rbase0 + it) * 272
                accs = [None] * 16
                for j in range(P):
                    for k in range(8):
                        word = plsc.load_gather(vtab, [vbase[j] + (wc + k)])
                        lo = plsc.bitcast(word << 16, f32)
                        hi = plsc.bitcast(word & HI, f32)
                        if accs[2 * k] is None:
                            accs[2 * k] = w[0] * lo
                            accs[2 * k + 1] = w[0] * hi
                        else:
                            accs[2 * k] = accs[2 * k] + w[j] * lo
                            accs[2 * k + 1] = accs[2 * k + 1] + w[j] * hi
                for d in range(16):
                    plsc.store_scatter(tr_v, [rbase + iota * 17 + d], accs[d])
                for k in range(16):
                    out_v[pl.ds((lbase + k) * 256 + col0 + it * 16, 16)] = (
                        tr_v[pl.ds(rbase + k * 17, 16)])

        def do_group(gi, crow):
            rows = crow + gi * _L + iota
            lbase = gi * _L

            rb = rows * 27
            ids = [plsc.load_gather(ids_v, [rb + j]) for j in range(8)]
            masks = [idj != 0 for idj in ids]
            pool_write(ids, masks, sh_v, vh_v, 33, 64, 0, lbase, 0)

            cards = [plsc.load_gather(ids_v, [rb + 8 + j]) for j in range(5)]
            picks = [plsc.load_gather(ids_v, [rb + 13 + j]) for j in range(5)]
            parts = [plsc.load_gather(ids_v, [rb + 18 + j]) for j in range(5)]
            tix = [cards[j] + 34 * (picks[j] + 2 * parts[j]) + 136 * j
                   for j in range(5)]
            masks = [cj != 0 for cj in cards]
            pool_write(tix, masks, stk_v, vt_v, 33, 64, 64, lbase, 4)

            ids = [plsc.load_gather(ids_v, [rb + 23 + j]) for j in range(2)]
            masks = [idj != 0 for idj in ids]
            pool_write(ids, masks, sb_v, vb_v, 17, 32, 128, lbase, 8)
            ids = [plsc.load_gather(ids_v, [rb + 25 + j]) for j in range(2)]
            masks = [idj != 0 for idj in ids]
            pool_write(ids, masks, sy_v, vy_v, 17, 32, 160, lbase, 10)

            @plsc.parallel_loop(0, 16, 1, unroll=4)
            def _hrow(k):
                for dc in range(0, 64, 16):
                    out_v[pl.ds((lbase + k) * 256 + 192 + dc, 16)] = (
                        plsc.bitcast(hdr_v[pl.ds((lbase + k) * 64 + dc, 16)],
                                     f32))
            return crow

        pltpu.async_copy(hdr_h.at[pl.ds(base * 64, CH * 64)], hdr_v, sem_h)

        def do_chunk(ci, carry):
            crow = ci * CH
            pltpu.make_async_copy(
                hdr_h.at[pl.ds((base + crow) * 64, CH * 64)], hdr_v,
                sem_h).wait()
            lax.fori_loop(0, GPC, do_group, crow)

            @pl.when(ci < NCH - 1)
            def _prefetch():
                pltpu.async_copy(
                    hdr_h.at[pl.ds((base + crow + CH) * 64, CH * 64)],
                    hdr_v, sem_h)

            pltpu.sync_copy(out_v,
                            out_h.at[pl.ds((base + crow) * 256, CH * 256)])
            return carry

        lax.fori_loop(0, NCH, do_chunk, 0)

    return sc_fn


def kernel(header_scalar, called_ids, hand_ids, blind_ids, bury_ids,
           trick_card_ids, trick_is_picker, trick_is_partner_known,
           card_table, seat_table, role_table,
           W_trick, b_trick, W_simple, b_simple,
           q_hand, Wk_hand, Wv_hand, q_trick, Wk_trick, Wv_trick,
           q_blind, Wk_blind, Wv_blind, q_bury, Wk_bury, Wv_bury,
           W_head, b_head):
    B = header_scalar.shape[0]
    i32 = jnp.int32

    called2 = called_ids.reshape(B, 1).astype(i32)
    sh, sb, sy, stk, vh, vt, vb, vy, hdr = _run_tables(
        B, header_scalar, called2, card_table, seat_table, role_table,
        W_trick, b_trick, W_simple, b_simple,
        q_hand, Wk_hand, Wv_hand, q_trick, Wk_trick, Wv_trick,
        q_blind, Wk_blind, Wv_blind, q_bury, Wk_bury, Wv_bury,
        W_head, b_head)

    def bc(x):
        return jax.lax.bitcast_convert_type(x, i32)

    def pack_v(v):
        T, Dv = v.shape
        w = jax.lax.bitcast_convert_type(
            v.astype(jnp.bfloat16).reshape(T, Dv // 2, 2), i32)
        return jnp.pad(w, ((0, 0), (0, 1))).reshape(-1)

    def sec(x, L):
        return jnp.pad(x, (0, L - x.shape[0]))

    tab_blob = jnp.concatenate([
        sec(bc(sh.reshape(-1)), 40),
        sec(bc(sb.reshape(-1)), 40),
        sec(bc(sy.reshape(-1)), 40),
        bc(stk.reshape(-1)),
        sec(pack_v(vh), 1128),
        pack_v(vt),
        sec(pack_v(vb), 584),
        sec(pack_v(vy), 584),
    ])
    ids27 = jnp.concatenate(
        [hand_ids, trick_card_ids, trick_is_picker, trick_is_partner_known,
         blind_ids, bury_ids], axis=1).astype(i32)
    out_flat = _make_sc(B)(ids27.reshape(-1), bc(hdr.reshape(-1)), tab_blob)
    return out_flat.reshape(B, 256)
